# baseline (speedup 1.0000x reference)
import functools

import jax
import jax.numpy as jnp
from jax import lax
from jax.experimental import pallas as pl

_ACTS = ("relu", "tanh", "swish", "gelu")
_BLOCK_B = 2048
_DN_T = (((1,), (1,)), ((), ()))
_DN_TN = (((0,), (0,)), ((), ()))
_DN_NN = (((1,), (0,)), ((), ()))


def _apply_act(name, h):
    if name == "relu":
        return jnp.maximum(h, 0.0)
    if name == "tanh":
        return jnp.tanh(h)
    if name == "swish":
        return h * (0.5 + 0.5 * jnp.tanh(0.5 * h))
    return 0.5 * h * (1.0 + jnp.tanh(0.7978845608028654
                                     * (h + 0.044715 * h * h * h)))


def _ensemble_body(x_ref, ew_ref, tf_ref, bl_ref, *wb_refs,
                   depths, acts, state_dim):
    o_ref = wb_refs[-1]
    wb_refs = wb_refs[:-1]
    x = x_ref[...]
    ew = ew_ref[...]
    hs = []
    wls = []
    k = 0
    for i in range(len(depths)):
        w0t = wb_refs[k][...]
        b0 = wb_refs[k + 1][...]
        k += 2
        tfi = tf_ref[:, i:i + 1]
        h = lax.dot_general(x, w0t[:state_dim, :], _DN_NN,
                            preferred_element_type=jnp.float32)
        trow = lax.dot_general(tfi, w0t[state_dim:, :], _DN_TN,
                               preferred_element_type=jnp.float32)
        h = _apply_act(acts[i], h + (b0 + trow))
        for j in range(1, depths[i] - 1):
            w = wb_refs[k][...]
            b = wb_refs[k + 1][...]
            k += 2
            h = _apply_act(acts[i],
                           lax.dot_general(h, w, _DN_T,
                                           preferred_element_type=jnp.float32)
                           + b)
        hs.append(ew[:, i:i + 1] * h)
        wls.append(wb_refs[k][...])
        k += 1
    hcat = jnp.concatenate(hs, axis=1)
    wcat = jnp.concatenate(wls, axis=1)
    out = lax.dot_general(hcat, wcat, _DN_T,
                          preferred_element_type=jnp.float32)
    outb = lax.dot_general(ew, bl_ref[...], _DN_NN,
                           preferred_element_type=jnp.float32)
    o_ref[...] = out + outb


def kernel(t, x, expert_weights, params, omegas):
    batch, state_dim = x.shape
    n_exp = len(params)
    depths = tuple(len(p) for p in params)
    acts = tuple(_ACTS[i % len(_ACTS)] for i in range(n_exp))

    tb = t[0]
    tf = jnp.stack([jnp.broadcast_to(tb, (n_exp,)),
                    jnp.sin(omegas * tb),
                    jnp.cos(omegas * tb)], axis=0)
    bl = jnp.stack([layers[-1]["b"] for layers in params])

    wb = []
    wb_specs = []
    for layers in params:
        last = len(layers) - 1
        for j, lyr in enumerate(layers):
            w = lyr["W"].T if j == 0 else lyr["W"]
            wb.append(w)
            wb_specs.append(pl.BlockSpec(w.shape, lambda i: (0, 0)))
            if j < last:
                b = lyr["b"].reshape(1, -1)
                wb.append(b)
                wb_specs.append(pl.BlockSpec(b.shape, lambda i: (0, 0)))

    blk = min(_BLOCK_B, batch)
    grid = (batch // blk,)
    body = functools.partial(_ensemble_body, depths=depths, acts=acts,
                             state_dim=state_dim)
    return pl.pallas_call(
        body,
        grid=grid,
        in_specs=[
            pl.BlockSpec((blk, state_dim), lambda i: (i, 0)),
            pl.BlockSpec((blk, n_exp), lambda i: (i, 0)),
            pl.BlockSpec(tf.shape, lambda i: (0, 0)),
            pl.BlockSpec(bl.shape, lambda i: (0, 0)),
        ] + wb_specs,
        out_specs=pl.BlockSpec((blk, state_dim), lambda i: (i, 0)),
        out_shape=jax.ShapeDtypeStruct((batch, state_dim), jnp.float32),
    )(x, expert_weights, tf, bl, *wb)


# speedup vs baseline: 1.1585x; 1.1585x over previous
import functools

import jax
import jax.numpy as jnp
from jax import lax
from jax.experimental import pallas as pl

_ACTS = ("relu", "tanh", "swish", "gelu")
_BLOCK_B = 2048
_DN_T = (((1,), (1,)), ((), ()))
_DN_TN = (((0,), (0,)), ((), ()))
_DN_NN = (((1,), (0,)), ((), ()))


def _apply_act(name, h):
    if name == "relu":
        return jnp.maximum(h, 0.0)
    if name == "tanh":
        return jnp.tanh(h)
    if name == "swish":
        return h * (0.5 + 0.5 * jnp.tanh(0.5 * h))
    return 0.5 * h * (1.0 + jnp.tanh(0.7978845608028654
                                     * (h + 0.044715 * h * h * h)))


def _ensemble_body(xt_ref, ewt_ref, tf_ref, bl_ref, *wb_refs,
                   depths, acts, state_dim):
    o_ref = wb_refs[-1]
    wb_refs = wb_refs[:-1]
    xt = xt_ref[...]
    ew = jnp.swapaxes(ewt_ref[...], 0, 1)
    hs = []
    wls = []
    k = 0
    for i in range(len(depths)):
        w0t = wb_refs[k][...]
        b0 = wb_refs[k + 1][...]
        k += 2
        tfi = tf_ref[:, i:i + 1]
        h = lax.dot_general(xt, w0t[:state_dim, :], _DN_TN,
                            preferred_element_type=jnp.float32)
        trow = lax.dot_general(tfi, w0t[state_dim:, :], _DN_TN,
                               preferred_element_type=jnp.float32)
        h = _apply_act(acts[i], h + (b0 + trow))
        for j in range(1, depths[i] - 1):
            w = wb_refs[k][...]
            b = wb_refs[k + 1][...]
            k += 2
            h = _apply_act(acts[i],
                           lax.dot_general(h, w, _DN_T,
                                           preferred_element_type=jnp.float32)
                           + b)
        hs.append(ew[:, i:i + 1] * h)
        wls.append(wb_refs[k][...])
        k += 1
    hcat = jnp.concatenate(hs, axis=1)
    wcat = jnp.concatenate(wls, axis=1)
    out = lax.dot_general(hcat, wcat, _DN_T,
                          preferred_element_type=jnp.float32)
    outb = lax.dot_general(ew, bl_ref[...], _DN_NN,
                           preferred_element_type=jnp.float32)
    o_ref[...] = out + outb


def kernel(t, x, expert_weights, params, omegas):
    batch, state_dim = x.shape
    n_exp = len(params)
    depths = tuple(len(p) for p in params)
    acts = tuple(_ACTS[i % len(_ACTS)] for i in range(n_exp))

    tb = t[0]
    tf = jnp.stack([jnp.broadcast_to(tb, (n_exp,)),
                    jnp.sin(omegas * tb),
                    jnp.cos(omegas * tb)], axis=0)
    bl = jnp.stack([layers[-1]["b"] for layers in params])

    wb = []
    wb_specs = []
    for layers in params:
        last = len(layers) - 1
        for j, lyr in enumerate(layers):
            w = lyr["W"].T if j == 0 else lyr["W"]
            wb.append(w)
            wb_specs.append(pl.BlockSpec(w.shape, lambda i: (0, 0)))
            if j < last:
                b = lyr["b"].reshape(1, -1)
                wb.append(b)
                wb_specs.append(pl.BlockSpec(b.shape, lambda i: (0, 0)))

    xt = x.T
    ewt = expert_weights.T

    blk = min(_BLOCK_B, batch)
    grid = (batch // blk,)
    body = functools.partial(_ensemble_body, depths=depths, acts=acts,
                             state_dim=state_dim)
    return pl.pallas_call(
        body,
        grid=grid,
        in_specs=[
            pl.BlockSpec((state_dim, blk), lambda i: (0, i)),
            pl.BlockSpec((n_exp, blk), lambda i: (0, i)),
            pl.BlockSpec(tf.shape, lambda i: (0, 0)),
            pl.BlockSpec(bl.shape, lambda i: (0, 0)),
        ] + wb_specs,
        out_specs=pl.BlockSpec((blk, state_dim), lambda i: (i, 0)),
        out_shape=jax.ShapeDtypeStruct((batch, state_dim), jnp.float32),
    )(xt, ewt, tf, bl, *wb)


# speedup vs baseline: 1.2497x; 1.0787x over previous
import functools

import jax
import jax.numpy as jnp
from jax import lax
from jax.experimental import pallas as pl

_ACTS = ("relu", "tanh", "swish", "gelu")
_BLOCK_B = 2048
_DN_T = (((1,), (1,)), ((), ()))
_DN_TN = (((0,), (0,)), ((), ()))
_DN_NN = (((1,), (0,)), ((), ()))


def _apply_act(name, h):
    if name == "relu":
        return jnp.maximum(h, 0.0)
    if name == "tanh":
        return jnp.tanh(h)
    if name == "swish":
        return h * (0.5 + 0.5 * jnp.tanh(0.5 * h))
    return 0.5 * h * (1.0 + jnp.tanh(0.7978845608028654
                                     * (h + 0.044715 * h * h * h)))


def _ensemble_body(xt_ref, ewt_ref, tf_ref, bl_ref, *wb_refs,
                   depths, acts, state_dim):
    o_ref = wb_refs[-1]
    wb_refs = wb_refs[:-1]
    xt = xt_ref[...]
    ew = jnp.swapaxes(ewt_ref[...], 0, 1)
    hs = []
    wls = []
    k = 0
    for i in range(len(depths)):
        w0t = wb_refs[k][...]
        b0 = wb_refs[k + 1][...]
        k += 2
        tfi = tf_ref[:, i:i + 1]
        h = lax.dot_general(xt, w0t[:state_dim, :], _DN_TN,
                            preferred_element_type=jnp.float32)
        trow = lax.dot_general(tfi, w0t[state_dim:, :], _DN_TN,
                               preferred_element_type=jnp.float32)
        h = _apply_act(acts[i], h + (b0 + trow))
        for j in range(1, depths[i] - 1):
            w = wb_refs[k][...]
            b = wb_refs[k + 1][...]
            k += 2
            h = _apply_act(acts[i],
                           lax.dot_general(h, w, _DN_T,
                                           preferred_element_type=jnp.float32)
                           + b)
        hs.append(ew[:, i:i + 1] * h)
        wls.append(wb_refs[k][...])
        k += 1
    hcat = jnp.concatenate(hs, axis=1)
    wcat = jnp.concatenate(wls, axis=1)
    out = lax.dot_general(hcat, wcat, _DN_T,
                          preferred_element_type=jnp.float32)
    outb = lax.dot_general(ew, bl_ref[...], _DN_NN,
                           preferred_element_type=jnp.float32)
    o_ref[...] = jnp.swapaxes(out + outb, 0, 1)


def kernel(t, x, expert_weights, params, omegas):
    batch, state_dim = x.shape
    n_exp = len(params)
    depths = tuple(len(p) for p in params)
    acts = tuple(_ACTS[i % len(_ACTS)] for i in range(n_exp))

    tb = t[0]
    tf = jnp.stack([jnp.broadcast_to(tb, (n_exp,)),
                    jnp.sin(omegas * tb),
                    jnp.cos(omegas * tb)], axis=0)
    bl = jnp.stack([layers[-1]["b"] for layers in params])

    wb = []
    wb_specs = []
    for layers in params:
        last = len(layers) - 1
        for j, lyr in enumerate(layers):
            w = lyr["W"].T if j == 0 else lyr["W"]
            wb.append(w)
            wb_specs.append(pl.BlockSpec(w.shape, lambda i: (0, 0)))
            if j < last:
                b = lyr["b"].reshape(1, -1)
                wb.append(b)
                wb_specs.append(pl.BlockSpec(b.shape, lambda i: (0, 0)))

    xt = x.T
    ewt = expert_weights.T

    blk = min(_BLOCK_B, batch)
    grid = (batch // blk,)
    body = functools.partial(_ensemble_body, depths=depths, acts=acts,
                             state_dim=state_dim)
    return pl.pallas_call(
        body,
        grid=grid,
        in_specs=[
            pl.BlockSpec((state_dim, blk), lambda i: (0, i)),
            pl.BlockSpec((n_exp, blk), lambda i: (0, i)),
            pl.BlockSpec(tf.shape, lambda i: (0, 0)),
            pl.BlockSpec(bl.shape, lambda i: (0, 0)),
        ] + wb_specs,
        out_specs=pl.BlockSpec((state_dim, blk), lambda i: (0, i)),
        out_shape=jax.ShapeDtypeStruct((state_dim, batch), jnp.float32),
    )(xt, ewt, tf, bl, *wb).T


# speedup vs baseline: 1.3338x; 1.0673x over previous
import functools

import jax
import jax.numpy as jnp
from jax import lax
from jax.experimental import pallas as pl

_ACTS = ("relu", "tanh", "swish", "gelu")
_BLOCK_B = 4096
_DN_T = (((1,), (1,)), ((), ()))
_DN_TN = (((0,), (0,)), ((), ()))
_DN_NN = (((1,), (0,)), ((), ()))


def _apply_act(name, h):
    if name == "relu":
        return jnp.maximum(h, 0.0)
    if name == "tanh":
        return jnp.tanh(h)
    if name == "swish":
        return h * (0.5 + 0.5 * jnp.tanh(0.5 * h))
    return 0.5 * h * (1.0 + jnp.tanh(0.7978845608028654
                                     * (h + 0.044715 * h * h * h)))


def _ensemble_body(xt_ref, ewt_ref, tf_ref, bl_ref, *wb_refs,
                   depths, acts, state_dim):
    o_ref = wb_refs[-1]
    wb_refs = wb_refs[:-1]
    xt = xt_ref[...]
    ew = jnp.swapaxes(ewt_ref[...], 0, 1)
    hs = []
    wls = []
    k = 0
    for i in range(len(depths)):
        w0t = wb_refs[k][...]
        b0 = wb_refs[k + 1][...]
        k += 2
        tfi = tf_ref[:, i:i + 1]
        h = lax.dot_general(xt, w0t[:state_dim, :], _DN_TN,
                            preferred_element_type=jnp.float32)
        trow = lax.dot_general(tfi, w0t[state_dim:, :], _DN_TN,
                               preferred_element_type=jnp.float32)
        h = _apply_act(acts[i], h + (b0 + trow))
        for j in range(1, depths[i] - 1):
            w = wb_refs[k][...]
            b = wb_refs[k + 1][...]
            k += 2
            h = _apply_act(acts[i],
                           lax.dot_general(h, w, _DN_T,
                                           preferred_element_type=jnp.float32)
                           + b)
        hs.append(ew[:, i:i + 1] * h)
        wls.append(wb_refs[k][...])
        k += 1
    hcat = jnp.concatenate(hs, axis=1)
    wcat = jnp.concatenate(wls, axis=1)
    out = lax.dot_general(hcat, wcat, _DN_T,
                          preferred_element_type=jnp.float32)
    outb = lax.dot_general(ew, bl_ref[...], _DN_NN,
                           preferred_element_type=jnp.float32)
    o_ref[...] = jnp.swapaxes(out + outb, 0, 1)


def kernel(t, x, expert_weights, params, omegas):
    batch, state_dim = x.shape
    n_exp = len(params)
    depths = tuple(len(p) for p in params)
    acts = tuple(_ACTS[i % len(_ACTS)] for i in range(n_exp))

    tb = t[0]
    tf = jnp.stack([jnp.broadcast_to(tb, (n_exp,)),
                    jnp.sin(omegas * tb),
                    jnp.cos(omegas * tb)], axis=0)
    bl = jnp.stack([layers[-1]["b"] for layers in params])

    wb = []
    wb_specs = []
    for layers in params:
        last = len(layers) - 1
        for j, lyr in enumerate(layers):
            w = lyr["W"].T if j == 0 else lyr["W"]
            wb.append(w)
            wb_specs.append(pl.BlockSpec(w.shape, lambda i: (0, 0)))
            if j < last:
                b = lyr["b"].reshape(1, -1)
                wb.append(b)
                wb_specs.append(pl.BlockSpec(b.shape, lambda i: (0, 0)))

    xt = x.T
    ewt = expert_weights.T

    blk = min(_BLOCK_B, batch)
    grid = (batch // blk,)
    body = functools.partial(_ensemble_body, depths=depths, acts=acts,
                             state_dim=state_dim)
    return pl.pallas_call(
        body,
        grid=grid,
        in_specs=[
            pl.BlockSpec((state_dim, blk), lambda i: (0, i)),
            pl.BlockSpec((n_exp, blk), lambda i: (0, i)),
            pl.BlockSpec(tf.shape, lambda i: (0, 0)),
            pl.BlockSpec(bl.shape, lambda i: (0, 0)),
        ] + wb_specs,
        out_specs=pl.BlockSpec((state_dim, blk), lambda i: (0, i)),
        out_shape=jax.ShapeDtypeStruct((state_dim, batch), jnp.float32),
    )(xt, ewt, tf, bl, *wb).T
